# baseline (speedup 1.0000x reference)
import functools

import jax
import jax.numpy as jnp
from jax import lax
from jax.experimental import pallas as pl
from jax.experimental.pallas import tpu as pltpu
from jax.experimental.pallas import tpu_sc as plsc

_NC = 2
_NS = 16
_NW = _NC * _NS
_CT = 8


def kernel(x, pos_table):
    T, B, D = x.shape
    t_per_w = T // _NW
    n_chunks = t_per_w // _CT
    mesh = plsc.VectorSubcoreMesh(core_axis_name="c", subcore_axis_name="s")

    @functools.partial(
        pl.kernel,
        mesh=mesh,
        out_type=jax.ShapeDtypeStruct((T, B, D), jnp.float32),
        scratch_types=[
            pltpu.VMEM((_CT, B, D), jnp.float32),
            pltpu.VMEM((_CT, B, D), jnp.float32),
            pltpu.VMEM((_CT, D), jnp.float32),
            pltpu.VMEM((_CT, D), jnp.float32),
            pltpu.SemaphoreType.DMA,
            pltpu.SemaphoreType.DMA,
            pltpu.SemaphoreType.DMA,
            pltpu.SemaphoreType.DMA,
        ],
    )
    def sc_add(x_hbm, pos_hbm, out_hbm, xv0, xv1, pv0, pv1, si0, si1, so0, so1):
        wid = lax.axis_index("s") * _NC + lax.axis_index("c")
        base = wid * t_per_w
        bufs = ((xv0, pv0, si0, so0), (xv1, pv1, si1, so1))

        def start_in(ci, xvb, pvb, sib):
            t0 = base + ci * _CT
            pltpu.async_copy(x_hbm.at[pl.ds(t0, _CT)], xvb, sib)
            pltpu.async_copy(pos_hbm.at[pl.ds(t0, _CT)], pvb, sib)

        def wait_in(xvb, pvb, sib):
            pltpu.make_async_copy(x_hbm.at[pl.ds(base, _CT)], xvb, sib).wait()
            pltpu.make_async_copy(pos_hbm.at[pl.ds(base, _CT)], pvb, sib).wait()

        def start_out(ci, xvb, sob):
            t0 = base + ci * _CT
            pltpu.async_copy(xvb, out_hbm.at[pl.ds(t0, _CT)], sob)

        def wait_out(xvb, sob):
            pltpu.make_async_copy(xvb, out_hbm.at[pl.ds(base, _CT)], sob).wait()

        start_in(0, xv0, pv0, si0)

        def pair(g, carry):
            for b in (0, 1):
                xvb, pvb, sib, sob = bufs[b]
                xvn, pvn, sin, son = bufs[1 - b]
                ci = g * 2 + b
                wait_in(xvb, pvb, sib)

                @pl.when(ci >= 1)
                def _():
                    wait_out(xvn, son)

                @pl.when(ci + 1 < n_chunks)
                def _():
                    start_in(ci + 1, xvn, pvn, sin)

                @plsc.parallel_loop(0, _CT, 1, unroll=4)
                def _row(j):
                    for k in range(D // 16):
                        sl = pl.ds(k * 16, 16)
                        p = pvb[j, sl]
                        for bb in range(B):
                            plsc.addupdate(xvb.at[j, bb, sl], p)
                start_out(ci, xvb, sob)
            return carry

        lax.fori_loop(0, n_chunks // 2, pair, 0)
        wait_out(xv1, so1)

    return sc_add(x, pos_table)


# speedup vs baseline: 1.6613x; 1.6613x over previous
import functools

import jax
import jax.numpy as jnp
from jax import lax
from jax.experimental import pallas as pl
from jax.experimental.pallas import tpu as pltpu
from jax.experimental.pallas import tpu_sc as plsc

_NC = 2
_NS = 16
_NW = _NC * _NS
_CT = 8


def kernel(x, pos_table):
    T, B, D = x.shape
    t_per_w = T // _NW
    n_chunks = t_per_w // _CT
    mesh = plsc.VectorSubcoreMesh(core_axis_name="c", subcore_axis_name="s")

    @functools.partial(
        pl.kernel,
        mesh=mesh,
        out_type=jax.ShapeDtypeStruct((T, B, D), jnp.float32),
        scratch_types=[
            pltpu.VMEM((_CT, B, D), jnp.float32),
            pltpu.VMEM((_CT, B, D), jnp.float32),
            pltpu.VMEM((_CT, D), jnp.float32),
            pltpu.VMEM((_CT, D), jnp.float32),
            pltpu.SemaphoreType.DMA,
            pltpu.SemaphoreType.DMA,
            pltpu.SemaphoreType.DMA,
            pltpu.SemaphoreType.DMA,
        ],
    )
    def sc_add(x_hbm, pos_hbm, out_hbm, xv0, xv1, pv0, pv1, si0, si1, so0, so1):
        wid = lax.axis_index("s") * _NC + lax.axis_index("c")
        base = wid * t_per_w
        bufs = ((xv0, pv0, si0, so0), (xv1, pv1, si1, so1))

        def start_in(ci, xvb, pvb, sib):
            t0 = base + ci * _CT
            pltpu.async_copy(x_hbm.at[pl.ds(t0, _CT)], xvb, sib)
            pltpu.async_copy(pos_hbm.at[pl.ds(t0, _CT)], pvb, sib)

        def wait_in(xvb, pvb, sib):
            pltpu.make_async_copy(x_hbm.at[pl.ds(base, _CT)], xvb, sib).wait()
            pltpu.make_async_copy(pos_hbm.at[pl.ds(base, _CT)], pvb, sib).wait()

        def start_out(ci, xvb, sob):
            t0 = base + ci * _CT
            pltpu.async_copy(xvb, out_hbm.at[pl.ds(t0, _CT)], sob)

        def wait_out(xvb, sob):
            pltpu.make_async_copy(xvb, out_hbm.at[pl.ds(base, _CT)], sob).wait()

        start_in(0, xv0, pv0, si0)

        def pair(g, carry):
            for b in (0, 1):
                xvb, pvb, sib, sob = bufs[b]
                xvn, pvn, sin, son = bufs[1 - b]
                ci = g * 2 + b
                wait_in(xvb, pvb, sib)

                @pl.when(ci >= 1)
                def _():
                    wait_out(xvn, son)

                @pl.when(ci + 1 < n_chunks)
                def _():
                    start_in(ci + 1, xvn, pvn, sin)

                @plsc.parallel_loop(0, _CT, 1)
                def _row(j):
                    @plsc.parallel_loop(0, D, 16, unroll=8)
                    def _lane(k0):
                        sl = pl.ds(k0, 16)
                        p = pvb[j, sl]
                        for bb in range(B):
                            plsc.addupdate(xvb.at[j, bb, sl], p)
                start_out(ci, xvb, sob)
            return carry

        lax.fori_loop(0, n_chunks // 2, pair, 0)
        wait_out(xv1, so1)

    return sc_add(x, pos_table)
